# Initial kernel scaffold; baseline (speedup 1.0000x reference)
import jax
import jax.numpy as jnp
from jax.experimental import pallas as pl


def kernel(x_user, x_item, edge_index_ui, edge_index_iu, Wl_ui0, Wr_ui0, Wl_iu0, Wr_iu0, Wl_ui1, Wr_ui1, Wl_iu1, Wr_iu1, b_ui0, b_iu0, b_ui1, b_iu1, head_W, head_b):
    raise NotImplementedError("write your pallas kernel here")


# speedup vs baseline: 3.5453x; 3.5453x over previous
import jax
import jax.numpy as jnp
from jax import lax
from jax.experimental import pallas as pl
from jax.experimental.pallas import tpu as pltpu
from jax.experimental.pallas import tpu_sc as plsc

N = 10000
E = 320000
D = 128
NC = 2
NS = 16
NT = NC * NS
EPT = E // NT
B = 50
J = EPT // B
G = 8
NG = J // G
NRP = 640
NP = NS * NRP
ZB = 64
_ZC = NRP // ZB


def _init_acc(z128_hbm, acc_sh, zbuf_v, s, r0):
    for k in range(_ZC):
        pltpu.sync_copy(z128_hbm.at[pl.ds(s * NRP + k * ZB, ZB)], zbuf_v)
        pltpu.sync_copy(zbuf_v, acc_sh.at[pl.ds(r0 + k * ZB, ZB)])


def _flush_acc(acc_sh, acc_out, zbuf_v, tid, r0):
    for k in range(_ZC):
        pltpu.sync_copy(acc_sh.at[pl.ds(r0 + k * ZB, ZB)], zbuf_v)
        pltpu.sync_copy(zbuf_v, acc_out.at[pl.ds(tid * NRP + k * ZB, ZB)])


def _seg_body(x_hbm, src_hbm, dst_hbm, z128_hbm,
              acc_out, acc_sh, src_v, dst_v, rows_v, zbuf_v, sem, sem2):
    c = lax.axis_index("c")
    s = lax.axis_index("s")
    tid = c * NS + s
    r0 = s * NRP
    _init_acc(z128_hbm, acc_sh, zbuf_v, s, r0)
    plsc.subcore_barrier()

    def group(g, carry):
        off = pl.multiple_of(tid * J + g * G, 8)
        pltpu.sync_copy(src_hbm.at[pl.ds(off, G)], src_v)
        pltpu.sync_copy(dst_hbm.at[pl.ds(off, G)], dst_v)
        for j in range(G):
            pltpu.async_copy(x_hbm.at[src_v.at[j]], rows_v, sem).wait()
            pltpu.async_copy(rows_v, acc_sh.at[dst_v.at[j]], sem2,
                             add=True).wait()
        return carry

    lax.fori_loop(0, NG, group, 0)
    plsc.subcore_barrier()
    _flush_acc(acc_sh, acc_out, zbuf_v, tid, r0)


def _cnt_body(dst_hbm, ones_hbm, z128_hbm,
              acc_out, acc_sh, dst_v, rows_v, zbuf_v, sem2):
    c = lax.axis_index("c")
    s = lax.axis_index("s")
    tid = c * NS + s
    r0 = s * NRP
    _init_acc(z128_hbm, acc_sh, zbuf_v, s, r0)
    pltpu.sync_copy(ones_hbm, rows_v)
    plsc.subcore_barrier()

    def group(g, carry):
        off = pl.multiple_of(tid * J + g * G, 8)
        pltpu.sync_copy(dst_hbm.at[pl.ds(off, G)], dst_v)
        for j in range(G):
            pltpu.async_copy(rows_v, acc_sh.at[dst_v.at[j]], sem2,
                             add=True).wait()
        return carry

    lax.fori_loop(0, NG, group, 0)
    plsc.subcore_barrier()
    _flush_acc(acc_sh, acc_out, zbuf_v, tid, r0)


def _make_seg():
    mesh = plsc.VectorSubcoreMesh(core_axis_name="c", subcore_axis_name="s")
    return pl.kernel(
        _seg_body,
        out_type=(jax.ShapeDtypeStruct((NC * NP, D), jnp.float32),),
        mesh=mesh,
        scratch_types=(
            pltpu.VMEM_SHARED((NP, D), jnp.float32),
            pltpu.VMEM((G, B), jnp.int32),
            pltpu.VMEM((G, B), jnp.int32),
            pltpu.VMEM((B, D), jnp.float32),
            pltpu.VMEM((ZB, D), jnp.float32),
            pltpu.SemaphoreType.DMA,
            pltpu.SemaphoreType.DMA,
        ),
    )


def _make_cnt():
    mesh = plsc.VectorSubcoreMesh(core_axis_name="c", subcore_axis_name="s")
    return pl.kernel(
        _cnt_body,
        out_type=(jax.ShapeDtypeStruct((NC * NP, D), jnp.float32),),
        mesh=mesh,
        scratch_types=(
            pltpu.VMEM_SHARED((NP, D), jnp.float32),
            pltpu.VMEM((G, B), jnp.int32),
            pltpu.VMEM((B, D), jnp.float32),
            pltpu.VMEM((ZB, D), jnp.float32),
            pltpu.SemaphoreType.DMA,
        ),
    )


def _combine_body(acc_ref, cnt_ref, x_ref, wl_ref, wr_ref, b_ref, o_ref):
    a = acc_ref[0] + acc_ref[1]
    cnt = cnt_ref[:, 0:1] + cnt_ref[:, 1:2]
    agg = a * (1.0 / jnp.maximum(cnt, 1.0))
    h = jnp.dot(agg, wl_ref[...], preferred_element_type=jnp.float32)
    h = h + jnp.dot(x_ref[...], wr_ref[...], preferred_element_type=jnp.float32)
    h = h + b_ref[...]
    o_ref[...] = jnp.where(h >= 0, h, 0.01 * h)


def _final_body(acc_ref, cnt_ref, x_ref, wl_ref, wr_ref, b_ref, hw_ref,
                hb_ref, o_ref):
    a = acc_ref[0] + acc_ref[1]
    cnt = cnt_ref[:, 0:1] + cnt_ref[:, 1:2]
    agg = a * (1.0 / jnp.maximum(cnt, 1.0))
    h = jnp.dot(agg, wl_ref[...], preferred_element_type=jnp.float32)
    h = h + jnp.dot(x_ref[...], wr_ref[...], preferred_element_type=jnp.float32)
    h = h + b_ref[...]
    h = jnp.where(h >= 0, h, 0.01 * h)
    o_ref[...] = (
        jnp.dot(h, hw_ref[...], preferred_element_type=jnp.float32)
        + hb_ref[...]
    )


_BLK = 1000


def _combine(acc, cnt, x, wl, wr, b):
    grid = (N // _BLK,)
    return pl.pallas_call(
        _combine_body,
        grid=grid,
        in_specs=[
            pl.BlockSpec((NC, _BLK, D), lambda i: (0, i, 0)),
            pl.BlockSpec((_BLK, NC), lambda i: (i, 0)),
            pl.BlockSpec((_BLK, D), lambda i: (i, 0)),
            pl.BlockSpec((D, D), lambda i: (0, 0)),
            pl.BlockSpec((D, D), lambda i: (0, 0)),
            pl.BlockSpec((1, D), lambda i: (0, 0)),
        ],
        out_specs=pl.BlockSpec((_BLK, D), lambda i: (i, 0)),
        out_shape=jax.ShapeDtypeStruct((N, D), jnp.float32),
    )(acc, cnt, x, wl, wr, b.reshape(1, D))


def _final(acc, cnt, x, wl, wr, b, hw, hb):
    grid = (N // _BLK,)
    nout = hw.shape[1]
    return pl.pallas_call(
        _final_body,
        grid=grid,
        in_specs=[
            pl.BlockSpec((NC, _BLK, D), lambda i: (0, i, 0)),
            pl.BlockSpec((_BLK, NC), lambda i: (i, 0)),
            pl.BlockSpec((_BLK, D), lambda i: (i, 0)),
            pl.BlockSpec((D, D), lambda i: (0, 0)),
            pl.BlockSpec((D, D), lambda i: (0, 0)),
            pl.BlockSpec((1, D), lambda i: (0, 0)),
            pl.BlockSpec((D, nout), lambda i: (0, 0)),
            pl.BlockSpec((1, nout), lambda i: (0, 0)),
        ],
        out_specs=pl.BlockSpec((_BLK, nout), lambda i: (i, 0)),
        out_shape=jax.ShapeDtypeStruct((N, nout), jnp.float32),
    )(acc, cnt, x, wl, wr, b.reshape(1, D), hw, hb.reshape(1, nout))


def kernel(x_user, x_item, edge_index_ui, edge_index_iu,
           Wl_ui0, Wr_ui0, Wl_iu0, Wr_iu0, Wl_ui1, Wr_ui1, Wl_iu1, Wr_iu1,
           b_ui0, b_iu0, b_ui1, b_iu1, head_W, head_b):
    src_ui = edge_index_ui[0].astype(jnp.int32).reshape(NT * J, B)
    dst_ui = edge_index_ui[1].astype(jnp.int32).reshape(NT * J, B)
    src_iu = edge_index_iu[0].astype(jnp.int32).reshape(NT * J, B)
    dst_iu = edge_index_iu[1].astype(jnp.int32).reshape(NT * J, B)
    z128 = jnp.zeros((NP, D), jnp.float32)
    ones = jnp.ones((B, D), jnp.float32)

    seg = _make_seg()
    cntseg = _make_cnt()

    def _racc(a):
        return a.reshape(NC, NP, D)

    def _rcnt(a):
        return a.reshape(NC, NP, D)[:, :N, 0].T

    (cacc_i,) = cntseg(dst_ui, ones, z128)
    (cacc_u,) = cntseg(dst_iu, ones, z128)
    (acc_i,) = seg(x_user, src_ui, dst_ui, z128)
    (acc_u,) = seg(x_item, src_iu, dst_iu, z128)
    cnt_i = _rcnt(cacc_i)
    cnt_u = _rcnt(cacc_u)
    h_item = _combine(_racc(acc_i), cnt_i, x_item, Wl_ui0, Wr_ui0, b_ui0)
    h_user = _combine(_racc(acc_u), cnt_u, x_user, Wl_iu0, Wr_iu0, b_iu0)
    (acc2,) = seg(h_item, src_iu, dst_iu, z128)
    return _final(_racc(acc2), cnt_u, h_user, Wl_iu1, Wr_iu1, b_iu1,
                  head_W, head_b)


# speedup vs baseline: 4.1322x; 1.1655x over previous
import jax
import jax.numpy as jnp
from jax import lax
from jax.experimental import pallas as pl
from jax.experimental.pallas import tpu as pltpu
from jax.experimental.pallas import tpu_sc as plsc

N = 10000
E = 320000
D = 128
NC = 2
NS = 16
NT = NC * NS
EPT = E // NT
B = 50
J = EPT // B
G = 8
NG = J // G
NRP = 640
NP = NS * NRP
ZB = 64
_ZC = NRP // ZB


def _init_acc(z128_hbm, acc_sh, zbuf_v, s, r0):
    for k in range(_ZC):
        pltpu.sync_copy(z128_hbm.at[pl.ds(s * NRP + k * ZB, ZB)], zbuf_v)
        pltpu.sync_copy(zbuf_v, acc_sh.at[pl.ds(r0 + k * ZB, ZB)])


def _flush_acc(acc_sh, acc_out, zbuf_v, tid, r0):
    for k in range(_ZC):
        pltpu.sync_copy(acc_sh.at[pl.ds(r0 + k * ZB, ZB)], zbuf_v)
        pltpu.sync_copy(zbuf_v, acc_out.at[pl.ds(tid * NRP + k * ZB, ZB)])


def _seg_body(x_hbm, src_hbm, dst_hbm, z128_hbm,
              acc_out, acc_sh, src_v, dst_v, rows_v, rows2_v, zbuf_v,
              sem, sem2):
    c = lax.axis_index("c")
    s = lax.axis_index("s")
    tid = c * NS + s
    r0 = s * NRP
    _init_acc(z128_hbm, acc_sh, zbuf_v, s, r0)
    plsc.subcore_barrier()

    rb = (rows_v, rows2_v)

    def group(g, carry):
        off = pl.multiple_of(tid * J + g * G, 8)
        pltpu.sync_copy(src_hbm.at[pl.ds(off, G)], src_v)
        pltpu.sync_copy(dst_hbm.at[pl.ds(off, G)], dst_v)
        pltpu.async_copy(x_hbm.at[src_v.at[0]], rb[0], sem).wait()
        for j in range(1, G):
            gd = pltpu.async_copy(x_hbm.at[src_v.at[j]], rb[j % 2], sem)
            sc = pltpu.async_copy(rb[(j - 1) % 2],
                                  acc_sh.at[dst_v.at[j - 1]], sem2,
                                  add=True)
            gd.wait()
            sc.wait()
        pltpu.async_copy(rb[(G - 1) % 2], acc_sh.at[dst_v.at[G - 1]],
                         sem2, add=True).wait()
        return carry

    lax.fori_loop(0, NG, group, 0)
    plsc.subcore_barrier()
    _flush_acc(acc_sh, acc_out, zbuf_v, tid, r0)


def _cnt_body(dst_hbm, ones_hbm, z128_hbm,
              acc_out, acc_sh, dst_v, rows_v, zbuf_v, sem2):
    c = lax.axis_index("c")
    s = lax.axis_index("s")
    tid = c * NS + s
    r0 = s * NRP
    _init_acc(z128_hbm, acc_sh, zbuf_v, s, r0)
    pltpu.sync_copy(ones_hbm, rows_v)
    plsc.subcore_barrier()

    def group(g, carry):
        off = pl.multiple_of(tid * J + g * G, 8)
        pltpu.sync_copy(dst_hbm.at[pl.ds(off, G)], dst_v)
        descs = [
            pltpu.async_copy(rows_v, acc_sh.at[dst_v.at[j]], sem2, add=True)
            for j in range(G)
        ]
        for d in descs:
            d.wait()
        return carry

    lax.fori_loop(0, NG, group, 0)
    plsc.subcore_barrier()
    _flush_acc(acc_sh, acc_out, zbuf_v, tid, r0)


def _make_seg():
    mesh = plsc.VectorSubcoreMesh(core_axis_name="c", subcore_axis_name="s")
    return pl.kernel(
        _seg_body,
        out_type=(jax.ShapeDtypeStruct((NC * NP, D), jnp.float32),),
        mesh=mesh,
        scratch_types=(
            pltpu.VMEM_SHARED((NP, D), jnp.float32),
            pltpu.VMEM((G, B), jnp.int32),
            pltpu.VMEM((G, B), jnp.int32),
            pltpu.VMEM((B, D), jnp.float32),
            pltpu.VMEM((B, D), jnp.float32),
            pltpu.VMEM((ZB, D), jnp.float32),
            pltpu.SemaphoreType.DMA,
            pltpu.SemaphoreType.DMA,
        ),
    )


def _make_cnt():
    mesh = plsc.VectorSubcoreMesh(core_axis_name="c", subcore_axis_name="s")
    return pl.kernel(
        _cnt_body,
        out_type=(jax.ShapeDtypeStruct((NC * NP, D), jnp.float32),),
        mesh=mesh,
        scratch_types=(
            pltpu.VMEM_SHARED((NP, D), jnp.float32),
            pltpu.VMEM((G, B), jnp.int32),
            pltpu.VMEM((B, D), jnp.float32),
            pltpu.VMEM((ZB, D), jnp.float32),
            pltpu.SemaphoreType.DMA,
        ),
    )


def _combine_body(acc_ref, cnt_ref, x_ref, wl_ref, wr_ref, b_ref, o_ref):
    a = acc_ref[0] + acc_ref[1]
    cnt = cnt_ref[:, 0:1] + cnt_ref[:, 1:2]
    agg = a * (1.0 / jnp.maximum(cnt, 1.0))
    h = jnp.dot(agg, wl_ref[...], preferred_element_type=jnp.float32)
    h = h + jnp.dot(x_ref[...], wr_ref[...], preferred_element_type=jnp.float32)
    h = h + b_ref[...]
    o_ref[...] = jnp.where(h >= 0, h, 0.01 * h)


def _final_body(acc_ref, cnt_ref, x_ref, wl_ref, wr_ref, b_ref, hw_ref,
                hb_ref, o_ref):
    a = acc_ref[0] + acc_ref[1]
    cnt = cnt_ref[:, 0:1] + cnt_ref[:, 1:2]
    agg = a * (1.0 / jnp.maximum(cnt, 1.0))
    h = jnp.dot(agg, wl_ref[...], preferred_element_type=jnp.float32)
    h = h + jnp.dot(x_ref[...], wr_ref[...], preferred_element_type=jnp.float32)
    h = h + b_ref[...]
    h = jnp.where(h >= 0, h, 0.01 * h)
    o_ref[...] = (
        jnp.dot(h, hw_ref[...], preferred_element_type=jnp.float32)
        + hb_ref[...]
    )


_BLK = 1000


def _combine(acc, cnt, x, wl, wr, b):
    grid = (N // _BLK,)
    return pl.pallas_call(
        _combine_body,
        grid=grid,
        in_specs=[
            pl.BlockSpec((NC, _BLK, D), lambda i: (0, i, 0)),
            pl.BlockSpec((_BLK, NC), lambda i: (i, 0)),
            pl.BlockSpec((_BLK, D), lambda i: (i, 0)),
            pl.BlockSpec((D, D), lambda i: (0, 0)),
            pl.BlockSpec((D, D), lambda i: (0, 0)),
            pl.BlockSpec((1, D), lambda i: (0, 0)),
        ],
        out_specs=pl.BlockSpec((_BLK, D), lambda i: (i, 0)),
        out_shape=jax.ShapeDtypeStruct((N, D), jnp.float32),
    )(acc, cnt, x, wl, wr, b.reshape(1, D))


def _final(acc, cnt, x, wl, wr, b, hw, hb):
    grid = (N // _BLK,)
    nout = hw.shape[1]
    return pl.pallas_call(
        _final_body,
        grid=grid,
        in_specs=[
            pl.BlockSpec((NC, _BLK, D), lambda i: (0, i, 0)),
            pl.BlockSpec((_BLK, NC), lambda i: (i, 0)),
            pl.BlockSpec((_BLK, D), lambda i: (i, 0)),
            pl.BlockSpec((D, D), lambda i: (0, 0)),
            pl.BlockSpec((D, D), lambda i: (0, 0)),
            pl.BlockSpec((1, D), lambda i: (0, 0)),
            pl.BlockSpec((D, nout), lambda i: (0, 0)),
            pl.BlockSpec((1, nout), lambda i: (0, 0)),
        ],
        out_specs=pl.BlockSpec((_BLK, nout), lambda i: (i, 0)),
        out_shape=jax.ShapeDtypeStruct((N, nout), jnp.float32),
    )(acc, cnt, x, wl, wr, b.reshape(1, D), hw, hb.reshape(1, nout))


def kernel(x_user, x_item, edge_index_ui, edge_index_iu,
           Wl_ui0, Wr_ui0, Wl_iu0, Wr_iu0, Wl_ui1, Wr_ui1, Wl_iu1, Wr_iu1,
           b_ui0, b_iu0, b_ui1, b_iu1, head_W, head_b):
    src_ui = edge_index_ui[0].astype(jnp.int32).reshape(NT * J, B)
    dst_ui = edge_index_ui[1].astype(jnp.int32).reshape(NT * J, B)
    src_iu = edge_index_iu[0].astype(jnp.int32).reshape(NT * J, B)
    dst_iu = edge_index_iu[1].astype(jnp.int32).reshape(NT * J, B)
    z128 = jnp.zeros((NP, D), jnp.float32)
    ones = jnp.ones((B, D), jnp.float32)

    seg = _make_seg()
    cntseg = _make_cnt()

    def _racc(a):
        return a.reshape(NC, NP, D)

    def _rcnt(a):
        return a.reshape(NC, NP, D)[:, :N, 0].T

    (cacc_i,) = cntseg(dst_ui, ones, z128)
    (cacc_u,) = cntseg(dst_iu, ones, z128)
    (acc_i,) = seg(x_user, src_ui, dst_ui, z128)
    (acc_u,) = seg(x_item, src_iu, dst_iu, z128)
    cnt_i = _rcnt(cacc_i)
    cnt_u = _rcnt(cacc_u)
    h_item = _combine(_racc(acc_i), cnt_i, x_item, Wl_ui0, Wr_ui0, b_ui0)
    h_user = _combine(_racc(acc_u), cnt_u, x_user, Wl_iu0, Wr_iu0, b_iu0)
    (acc2,) = seg(h_item, src_iu, dst_iu, z128)
    return _final(_racc(acc2), cnt_u, h_user, Wl_iu1, Wr_iu1, b_iu1,
                  head_W, head_b)


# speedup vs baseline: 5.8218x; 1.4089x over previous
import jax
import jax.numpy as jnp
from jax import lax
from jax.experimental import pallas as pl
from jax.experimental.pallas import tpu as pltpu
from jax.experimental.pallas import tpu_sc as plsc

N = 10000
E = 320000
D = 128
NC = 2
NS = 16
NT = NC * NS
EPT = E // NT
B = 125
J = EPT // B
G = 8
NG = J // G
NRP = 640
NP = NS * NRP
ZB = 64
_ZC = NRP // ZB


def _init_acc(z128_hbm, acc_sh, zbuf_v, s, r0):
    for k in range(_ZC):
        pltpu.sync_copy(z128_hbm.at[pl.ds(s * NRP + k * ZB, ZB)], zbuf_v)
        pltpu.sync_copy(zbuf_v, acc_sh.at[pl.ds(r0 + k * ZB, ZB)])


def _flush_acc(acc_sh, acc_out, zbuf_v, tid, r0):
    for k in range(_ZC):
        pltpu.sync_copy(acc_sh.at[pl.ds(r0 + k * ZB, ZB)], zbuf_v)
        pltpu.sync_copy(zbuf_v, acc_out.at[pl.ds(tid * NRP + k * ZB, ZB)])


def _seg_body(x_hbm, src_hbm, dst_hbm, z128_hbm,
              acc_out, acc_sh, src_v, dst_v, rows_v, rows2_v, zbuf_v,
              sem, sem2):
    c = lax.axis_index("c")
    s = lax.axis_index("s")
    tid = c * NS + s
    r0 = s * NRP
    _init_acc(z128_hbm, acc_sh, zbuf_v, s, r0)
    plsc.subcore_barrier()

    rb = (rows_v, rows2_v)

    def group(g, carry):
        off = pl.multiple_of(tid * J + g * G, 8)
        pltpu.sync_copy(src_hbm.at[pl.ds(off, G)], src_v)
        pltpu.sync_copy(dst_hbm.at[pl.ds(off, G)], dst_v)
        pltpu.async_copy(x_hbm.at[src_v.at[0]], rb[0], sem).wait()
        for j in range(1, G):
            gd = pltpu.async_copy(x_hbm.at[src_v.at[j]], rb[j % 2], sem)
            sc = pltpu.async_copy(rb[(j - 1) % 2],
                                  acc_sh.at[dst_v.at[j - 1]], sem2,
                                  add=True)
            gd.wait()
            sc.wait()
        pltpu.async_copy(rb[(G - 1) % 2], acc_sh.at[dst_v.at[G - 1]],
                         sem2, add=True).wait()
        return carry

    lax.fori_loop(0, NG, group, 0)
    plsc.subcore_barrier()
    _flush_acc(acc_sh, acc_out, zbuf_v, tid, r0)


def _cnt_body(dst_hbm, ones_hbm, z128_hbm,
              acc_out, acc_sh, dst_v, rows_v, zbuf_v, sem2):
    c = lax.axis_index("c")
    s = lax.axis_index("s")
    tid = c * NS + s
    r0 = s * NRP
    _init_acc(z128_hbm, acc_sh, zbuf_v, s, r0)
    pltpu.sync_copy(ones_hbm, rows_v)
    plsc.subcore_barrier()

    def group(g, carry):
        off = pl.multiple_of(tid * J + g * G, 8)
        pltpu.sync_copy(dst_hbm.at[pl.ds(off, G)], dst_v)
        descs = [
            pltpu.async_copy(rows_v, acc_sh.at[dst_v.at[j]], sem2, add=True)
            for j in range(G)
        ]
        for d in descs:
            d.wait()
        return carry

    lax.fori_loop(0, NG, group, 0)
    plsc.subcore_barrier()
    _flush_acc(acc_sh, acc_out, zbuf_v, tid, r0)


def _make_seg():
    mesh = plsc.VectorSubcoreMesh(core_axis_name="c", subcore_axis_name="s")
    return pl.kernel(
        _seg_body,
        out_type=(jax.ShapeDtypeStruct((NC * NP, D), jnp.float32),),
        mesh=mesh,
        scratch_types=(
            pltpu.VMEM_SHARED((NP, D), jnp.float32),
            pltpu.VMEM((G, B), jnp.int32),
            pltpu.VMEM((G, B), jnp.int32),
            pltpu.VMEM((B, D), jnp.float32),
            pltpu.VMEM((B, D), jnp.float32),
            pltpu.VMEM((ZB, D), jnp.float32),
            pltpu.SemaphoreType.DMA,
            pltpu.SemaphoreType.DMA,
        ),
    )


def _make_cnt():
    mesh = plsc.VectorSubcoreMesh(core_axis_name="c", subcore_axis_name="s")
    return pl.kernel(
        _cnt_body,
        out_type=(jax.ShapeDtypeStruct((NC * NP, D), jnp.float32),),
        mesh=mesh,
        scratch_types=(
            pltpu.VMEM_SHARED((NP, D), jnp.float32),
            pltpu.VMEM((G, B), jnp.int32),
            pltpu.VMEM((B, D), jnp.float32),
            pltpu.VMEM((ZB, D), jnp.float32),
            pltpu.SemaphoreType.DMA,
        ),
    )


def _combine_body(acc_ref, cnt_ref, x_ref, wl_ref, wr_ref, b_ref, o_ref):
    a = acc_ref[0] + acc_ref[1]
    cnt = cnt_ref[:, 0:1] + cnt_ref[:, 1:2]
    agg = a * (1.0 / jnp.maximum(cnt, 1.0))
    h = jnp.dot(agg, wl_ref[...], preferred_element_type=jnp.float32)
    h = h + jnp.dot(x_ref[...], wr_ref[...], preferred_element_type=jnp.float32)
    h = h + b_ref[...]
    o_ref[...] = jnp.where(h >= 0, h, 0.01 * h)


def _final_body(acc_ref, cnt_ref, x_ref, wl_ref, wr_ref, b_ref, hw_ref,
                hb_ref, o_ref):
    a = acc_ref[0] + acc_ref[1]
    cnt = cnt_ref[:, 0:1] + cnt_ref[:, 1:2]
    agg = a * (1.0 / jnp.maximum(cnt, 1.0))
    h = jnp.dot(agg, wl_ref[...], preferred_element_type=jnp.float32)
    h = h + jnp.dot(x_ref[...], wr_ref[...], preferred_element_type=jnp.float32)
    h = h + b_ref[...]
    h = jnp.where(h >= 0, h, 0.01 * h)
    o_ref[...] = (
        jnp.dot(h, hw_ref[...], preferred_element_type=jnp.float32)
        + hb_ref[...]
    )


_BLK = 1000


def _combine(acc, cnt, x, wl, wr, b):
    grid = (N // _BLK,)
    return pl.pallas_call(
        _combine_body,
        grid=grid,
        in_specs=[
            pl.BlockSpec((NC, _BLK, D), lambda i: (0, i, 0)),
            pl.BlockSpec((_BLK, NC), lambda i: (i, 0)),
            pl.BlockSpec((_BLK, D), lambda i: (i, 0)),
            pl.BlockSpec((D, D), lambda i: (0, 0)),
            pl.BlockSpec((D, D), lambda i: (0, 0)),
            pl.BlockSpec((1, D), lambda i: (0, 0)),
        ],
        out_specs=pl.BlockSpec((_BLK, D), lambda i: (i, 0)),
        out_shape=jax.ShapeDtypeStruct((N, D), jnp.float32),
    )(acc, cnt, x, wl, wr, b.reshape(1, D))


def _final(acc, cnt, x, wl, wr, b, hw, hb):
    grid = (N // _BLK,)
    nout = hw.shape[1]
    return pl.pallas_call(
        _final_body,
        grid=grid,
        in_specs=[
            pl.BlockSpec((NC, _BLK, D), lambda i: (0, i, 0)),
            pl.BlockSpec((_BLK, NC), lambda i: (i, 0)),
            pl.BlockSpec((_BLK, D), lambda i: (i, 0)),
            pl.BlockSpec((D, D), lambda i: (0, 0)),
            pl.BlockSpec((D, D), lambda i: (0, 0)),
            pl.BlockSpec((1, D), lambda i: (0, 0)),
            pl.BlockSpec((D, nout), lambda i: (0, 0)),
            pl.BlockSpec((1, nout), lambda i: (0, 0)),
        ],
        out_specs=pl.BlockSpec((_BLK, nout), lambda i: (i, 0)),
        out_shape=jax.ShapeDtypeStruct((N, nout), jnp.float32),
    )(acc, cnt, x, wl, wr, b.reshape(1, D), hw, hb.reshape(1, nout))


def kernel(x_user, x_item, edge_index_ui, edge_index_iu,
           Wl_ui0, Wr_ui0, Wl_iu0, Wr_iu0, Wl_ui1, Wr_ui1, Wl_iu1, Wr_iu1,
           b_ui0, b_iu0, b_ui1, b_iu1, head_W, head_b):
    src_ui = edge_index_ui[0].astype(jnp.int32).reshape(NT * J, B)
    dst_ui = edge_index_ui[1].astype(jnp.int32).reshape(NT * J, B)
    src_iu = edge_index_iu[0].astype(jnp.int32).reshape(NT * J, B)
    dst_iu = edge_index_iu[1].astype(jnp.int32).reshape(NT * J, B)
    z128 = jnp.zeros((NP, D), jnp.float32)
    ones = jnp.ones((B, D), jnp.float32)

    seg = _make_seg()
    cntseg = _make_cnt()

    def _racc(a):
        return a.reshape(NC, NP, D)

    def _rcnt(a):
        return a.reshape(NC, NP, D)[:, :N, 0].T

    (cacc_i,) = cntseg(dst_ui, ones, z128)
    (cacc_u,) = cntseg(dst_iu, ones, z128)
    (acc_i,) = seg(x_user, src_ui, dst_ui, z128)
    (acc_u,) = seg(x_item, src_iu, dst_iu, z128)
    cnt_i = _rcnt(cacc_i)
    cnt_u = _rcnt(cacc_u)
    h_item = _combine(_racc(acc_i), cnt_i, x_item, Wl_ui0, Wr_ui0, b_ui0)
    h_user = _combine(_racc(acc_u), cnt_u, x_user, Wl_iu0, Wr_iu0, b_iu0)
    (acc2,) = seg(h_item, src_iu, dst_iu, z128)
    return _final(_racc(acc2), cnt_u, h_user, Wl_iu1, Wr_iu1, b_iu1,
                  head_W, head_b)


# speedup vs baseline: 6.0332x; 1.0363x over previous
import jax
import jax.numpy as jnp
from jax import lax
from jax.experimental import pallas as pl
from jax.experimental.pallas import tpu as pltpu
from jax.experimental.pallas import tpu_sc as plsc

N = 10000
E = 320000
D = 128
NC = 2
NS = 16
NT = NC * NS
EPT = E // NT
B = 125
J = EPT // B
G = 16
NG = J // G
NRP = 640
NP = NS * NRP
ZB = 64
_ZC = NRP // ZB


def _init_acc(z128_hbm, acc_sh, zbuf_v, s, r0):
    for k in range(_ZC):
        pltpu.sync_copy(z128_hbm.at[pl.ds(s * NRP + k * ZB, ZB)], zbuf_v)
        pltpu.sync_copy(zbuf_v, acc_sh.at[pl.ds(r0 + k * ZB, ZB)])


def _flush_acc(acc_sh, acc_out, zbuf_v, tid, r0):
    for k in range(_ZC):
        pltpu.sync_copy(acc_sh.at[pl.ds(r0 + k * ZB, ZB)], zbuf_v)
        pltpu.sync_copy(zbuf_v, acc_out.at[pl.ds(tid * NRP + k * ZB, ZB)])


def _seg_body(x_hbm, src_hbm, dst_hbm, z128_hbm,
              acc_out, acc_sh, src_v, dst_v, rows_v, rows2_v, zbuf_v,
              sem, sem2):
    c = lax.axis_index("c")
    s = lax.axis_index("s")
    tid = c * NS + s
    r0 = s * NRP
    _init_acc(z128_hbm, acc_sh, zbuf_v, s, r0)
    plsc.subcore_barrier()

    rb = (rows_v, rows2_v)

    def group(g, carry):
        off = pl.multiple_of(tid * J + g * G, 8)
        pltpu.sync_copy(src_hbm.at[pl.ds(off, G)], src_v)
        pltpu.sync_copy(dst_hbm.at[pl.ds(off, G)], dst_v)
        pltpu.async_copy(x_hbm.at[src_v.at[0]], rb[0], sem).wait()
        for j in range(1, G):
            gd = pltpu.async_copy(x_hbm.at[src_v.at[j]], rb[j % 2], sem)
            sc = pltpu.async_copy(rb[(j - 1) % 2],
                                  acc_sh.at[dst_v.at[j - 1]], sem2,
                                  add=True)
            gd.wait()
            sc.wait()
        pltpu.async_copy(rb[(G - 1) % 2], acc_sh.at[dst_v.at[G - 1]],
                         sem2, add=True).wait()
        return carry

    lax.fori_loop(0, NG, group, 0)
    plsc.subcore_barrier()
    _flush_acc(acc_sh, acc_out, zbuf_v, tid, r0)


def _cnt_body(dst_hbm, ones_hbm, z128_hbm,
              acc_out, acc_sh, dst_v, rows_v, zbuf_v, sem2):
    c = lax.axis_index("c")
    s = lax.axis_index("s")
    tid = c * NS + s
    r0 = s * NRP
    _init_acc(z128_hbm, acc_sh, zbuf_v, s, r0)
    pltpu.sync_copy(ones_hbm, rows_v)
    plsc.subcore_barrier()

    def group(g, carry):
        off = pl.multiple_of(tid * J + g * G, 8)
        pltpu.sync_copy(dst_hbm.at[pl.ds(off, G)], dst_v)
        descs = [
            pltpu.async_copy(rows_v, acc_sh.at[dst_v.at[j]], sem2, add=True)
            for j in range(G)
        ]
        for d in descs:
            d.wait()
        return carry

    lax.fori_loop(0, NG, group, 0)
    plsc.subcore_barrier()
    _flush_acc(acc_sh, acc_out, zbuf_v, tid, r0)


def _make_seg():
    mesh = plsc.VectorSubcoreMesh(core_axis_name="c", subcore_axis_name="s")
    return pl.kernel(
        _seg_body,
        out_type=(jax.ShapeDtypeStruct((NC * NP, D), jnp.float32),),
        mesh=mesh,
        scratch_types=(
            pltpu.VMEM_SHARED((NP, D), jnp.float32),
            pltpu.VMEM((G, B), jnp.int32),
            pltpu.VMEM((G, B), jnp.int32),
            pltpu.VMEM((B, D), jnp.float32),
            pltpu.VMEM((B, D), jnp.float32),
            pltpu.VMEM((ZB, D), jnp.float32),
            pltpu.SemaphoreType.DMA,
            pltpu.SemaphoreType.DMA,
        ),
    )


def _make_cnt():
    mesh = plsc.VectorSubcoreMesh(core_axis_name="c", subcore_axis_name="s")
    return pl.kernel(
        _cnt_body,
        out_type=(jax.ShapeDtypeStruct((NC * NP, D), jnp.float32),),
        mesh=mesh,
        scratch_types=(
            pltpu.VMEM_SHARED((NP, D), jnp.float32),
            pltpu.VMEM((G, B), jnp.int32),
            pltpu.VMEM((B, D), jnp.float32),
            pltpu.VMEM((ZB, D), jnp.float32),
            pltpu.SemaphoreType.DMA,
        ),
    )


def _combine_body(acc_ref, cnt_ref, x_ref, wl_ref, wr_ref, b_ref, o_ref):
    a = acc_ref[0] + acc_ref[1]
    cnt = cnt_ref[:, 0:1] + cnt_ref[:, 1:2]
    agg = a * (1.0 / jnp.maximum(cnt, 1.0))
    h = jnp.dot(agg, wl_ref[...], preferred_element_type=jnp.float32)
    h = h + jnp.dot(x_ref[...], wr_ref[...], preferred_element_type=jnp.float32)
    h = h + b_ref[...]
    o_ref[...] = jnp.where(h >= 0, h, 0.01 * h)


def _final_body(acc_ref, cnt_ref, x_ref, wl_ref, wr_ref, b_ref, hw_ref,
                hb_ref, o_ref):
    a = acc_ref[0] + acc_ref[1]
    cnt = cnt_ref[:, 0:1] + cnt_ref[:, 1:2]
    agg = a * (1.0 / jnp.maximum(cnt, 1.0))
    h = jnp.dot(agg, wl_ref[...], preferred_element_type=jnp.float32)
    h = h + jnp.dot(x_ref[...], wr_ref[...], preferred_element_type=jnp.float32)
    h = h + b_ref[...]
    h = jnp.where(h >= 0, h, 0.01 * h)
    o_ref[...] = (
        jnp.dot(h, hw_ref[...], preferred_element_type=jnp.float32)
        + hb_ref[...]
    )


_BLK = 1000


def _combine(acc, cnt, x, wl, wr, b):
    grid = (N // _BLK,)
    return pl.pallas_call(
        _combine_body,
        grid=grid,
        in_specs=[
            pl.BlockSpec((NC, _BLK, D), lambda i: (0, i, 0)),
            pl.BlockSpec((_BLK, NC), lambda i: (i, 0)),
            pl.BlockSpec((_BLK, D), lambda i: (i, 0)),
            pl.BlockSpec((D, D), lambda i: (0, 0)),
            pl.BlockSpec((D, D), lambda i: (0, 0)),
            pl.BlockSpec((1, D), lambda i: (0, 0)),
        ],
        out_specs=pl.BlockSpec((_BLK, D), lambda i: (i, 0)),
        out_shape=jax.ShapeDtypeStruct((N, D), jnp.float32),
    )(acc, cnt, x, wl, wr, b.reshape(1, D))


def _final(acc, cnt, x, wl, wr, b, hw, hb):
    grid = (N // _BLK,)
    nout = hw.shape[1]
    return pl.pallas_call(
        _final_body,
        grid=grid,
        in_specs=[
            pl.BlockSpec((NC, _BLK, D), lambda i: (0, i, 0)),
            pl.BlockSpec((_BLK, NC), lambda i: (i, 0)),
            pl.BlockSpec((_BLK, D), lambda i: (i, 0)),
            pl.BlockSpec((D, D), lambda i: (0, 0)),
            pl.BlockSpec((D, D), lambda i: (0, 0)),
            pl.BlockSpec((1, D), lambda i: (0, 0)),
            pl.BlockSpec((D, nout), lambda i: (0, 0)),
            pl.BlockSpec((1, nout), lambda i: (0, 0)),
        ],
        out_specs=pl.BlockSpec((_BLK, nout), lambda i: (i, 0)),
        out_shape=jax.ShapeDtypeStruct((N, nout), jnp.float32),
    )(acc, cnt, x, wl, wr, b.reshape(1, D), hw, hb.reshape(1, nout))


def kernel(x_user, x_item, edge_index_ui, edge_index_iu,
           Wl_ui0, Wr_ui0, Wl_iu0, Wr_iu0, Wl_ui1, Wr_ui1, Wl_iu1, Wr_iu1,
           b_ui0, b_iu0, b_ui1, b_iu1, head_W, head_b):
    src_ui = edge_index_ui[0].astype(jnp.int32).reshape(NT * J, B)
    dst_ui = edge_index_ui[1].astype(jnp.int32).reshape(NT * J, B)
    src_iu = edge_index_iu[0].astype(jnp.int32).reshape(NT * J, B)
    dst_iu = edge_index_iu[1].astype(jnp.int32).reshape(NT * J, B)
    z128 = jnp.zeros((NP, D), jnp.float32)
    ones = jnp.ones((B, D), jnp.float32)

    seg = _make_seg()
    cntseg = _make_cnt()

    def _racc(a):
        return a.reshape(NC, NP, D)

    def _rcnt(a):
        return a.reshape(NC, NP, D)[:, :N, 0].T

    (cacc_i,) = cntseg(dst_ui, ones, z128)
    (cacc_u,) = cntseg(dst_iu, ones, z128)
    (acc_i,) = seg(x_user, src_ui, dst_ui, z128)
    (acc_u,) = seg(x_item, src_iu, dst_iu, z128)
    cnt_i = _rcnt(cacc_i)
    cnt_u = _rcnt(cacc_u)
    h_item = _combine(_racc(acc_i), cnt_i, x_item, Wl_ui0, Wr_ui0, b_ui0)
    h_user = _combine(_racc(acc_u), cnt_u, x_user, Wl_iu0, Wr_iu0, b_iu0)
    (acc2,) = seg(h_item, src_iu, dst_iu, z128)
    return _final(_racc(acc2), cnt_u, h_user, Wl_iu1, Wr_iu1, b_iu1,
                  head_W, head_b)
